# Initial kernel scaffold; baseline (speedup 1.0000x reference)
import jax
import jax.numpy as jnp
from jax.experimental import pallas as pl


def kernel(x, target):
    raise NotImplementedError("write your pallas kernel here")


# speedup vs baseline: 5.8909x; 5.8909x over previous
import math

import jax
import jax.numpy as jnp
from jax import lax
from jax.experimental import pallas as pl
from jax.experimental.pallas import tpu as pltpu

VOCAB = 32000
PAD = 0
SMOOTH = 0.1
CONF = 1.0 - SMOOTH
EPS = SMOOTH / (VOCAB - 1)
KCONST = (VOCAB - 1) * EPS * math.log(EPS) + CONF * math.log(CONF)

ROWS = 2048
R = 64
NB = ROWS // R


def _tc_body(x_ref, t_ref, out_ref, acc_s, acc_g, acc_n):
    i = pl.program_id(0)

    @pl.when(i == 0)
    def _init():
        acc_s[0] = 0.0
        acc_g[0] = 0.0
        acc_n[0] = 0.0

    xb = x_ref[...]
    tb = t_ref[0, 0, :]
    mask = tb != PAD
    maskf = mask.astype(jnp.float32)[:, None]
    xm = xb * maskf
    cols = lax.broadcasted_iota(jnp.int32, (R, VOCAB), 1)
    sel = (cols == tb[:, None]).astype(jnp.float32)

    acc_s[0] += jnp.sum(xm)
    acc_g[0] += jnp.sum(xm * sel)
    acc_n[0] += jnp.sum(maskf)

    @pl.when(i == NB - 1)
    def _fin():
        tok = acc_n[0]
        num = KCONST * tok - EPS * acc_s[0] + (EPS - CONF) * acc_g[0]
        out_ref[0, 0] = num / tok


@jax.jit
def _tc(x, t3):
    return pl.pallas_call(
        _tc_body,
        grid=(NB,),
        in_specs=[
            pl.BlockSpec((R, VOCAB), lambda i: (i, 0)),
            pl.BlockSpec((1, 1, R), lambda i: (i, 0, 0)),
        ],
        out_specs=pl.BlockSpec(memory_space=pltpu.SMEM),
        out_shape=jax.ShapeDtypeStruct((1, 1), jnp.float32),
        scratch_shapes=[
            pltpu.SMEM((1,), jnp.float32),
            pltpu.SMEM((1,), jnp.float32),
            pltpu.SMEM((1,), jnp.float32),
        ],
    )(x, t3)


def kernel(x, target):
    t3 = target.astype(jnp.int32).reshape(NB, 1, R)
    return _tc(x, t3)[0, 0]
